# Initial kernel scaffold; baseline (speedup 1.0000x reference)
import jax
import jax.numpy as jnp
from jax.experimental import pallas as pl


def kernel(x, edge_attr, Wn, bn, We, be, Wa, ba, W1, b1, W2, b2, W3, b3, edge_index):
    raise NotImplementedError("write your pallas kernel here")


# speedup vs baseline: 3.6348x; 3.6348x over previous
import functools

import jax
import jax.numpy as jnp
from jax import lax
from jax.experimental import pallas as pl
from jax.experimental.pallas import tpu as pltpu
from jax.experimental.pallas import tpu_sc as plsc

N, E, D, DE, H = 10000, 320000, 128, 16, 256
NC, NS = 2, 16
NW = NC * NS
EPW = E // NW
CH = 80
NCH = EPW // CH
NP = 10240
NPT = NP // NS

MB = 2000
NB = E // MB

_SC_MESH = plsc.VectorSubcoreMesh(core_axis_name="c", subcore_axis_name="s")


def _gather_body(x_hbm, row_hbm, col_hbm, g0_hbm, g1_hbm,
                 ibuf0, ibuf1, rows0, rows1, sem0, sem1):
    c = lax.axis_index("c")
    s = lax.axis_index("s")
    wid = s * NC + c
    base = wid * EPW

    def chunk(i, carry):
        e0 = base + i * CH
        pltpu.sync_copy(row_hbm.at[pl.ds(e0, CH)], ibuf0)
        pltpu.sync_copy(col_hbm.at[pl.ds(e0, CH)], ibuf1)
        cp0 = pltpu.async_copy(x_hbm.at[ibuf0], rows0, sem0)
        cp1 = pltpu.async_copy(x_hbm.at[ibuf1], rows1, sem1)
        cp0.wait()
        cp1.wait()
        pltpu.sync_copy(rows0, g0_hbm.at[pl.ds(e0, CH)])
        pltpu.sync_copy(rows1, g1_hbm.at[pl.ds(e0, CH)])
        return carry

    lax.fori_loop(0, NCH, chunk, 0)


_gather_call = functools.partial(
    pl.kernel,
    out_type=(jax.ShapeDtypeStruct((E, D), jnp.float32),
              jax.ShapeDtypeStruct((E, D), jnp.float32)),
    mesh=_SC_MESH,
    scratch_types=[
        pltpu.VMEM((CH,), jnp.int32),
        pltpu.VMEM((CH,), jnp.int32),
        pltpu.VMEM((CH, D), jnp.float32),
        pltpu.VMEM((CH, D), jnp.float32),
        pltpu.SemaphoreType.DMA,
        pltpu.SemaphoreType.DMA,
    ],
)(_gather_body)


def _scatter_body(wm_hbm, col_hbm, zero_hbm, part_hbm, acc, ibuf, wbuf):
    c = lax.axis_index("c")
    s = lax.axis_index("s")
    wid = s * NC + c
    base = wid * EPW
    r0 = s * NPT

    pltpu.sync_copy(zero_hbm.at[pl.ds(r0, NPT)], acc.at[pl.ds(r0, NPT)])
    plsc.subcore_barrier()

    def chunk(i, carry):
        e0 = base + i * CH
        pltpu.sync_copy(col_hbm.at[pl.ds(e0, CH)], ibuf)
        pltpu.sync_copy(wm_hbm.at[pl.ds(e0, CH)], wbuf)
        pltpu.sync_copy(wbuf, acc.at[ibuf], add=True)
        return carry

    lax.fori_loop(0, NCH, chunk, 0)
    plsc.subcore_barrier()
    pltpu.sync_copy(acc.at[pl.ds(r0, NPT)], part_hbm.at[c, pl.ds(r0, NPT)])


_scatter_call = functools.partial(
    pl.kernel,
    out_type=jax.ShapeDtypeStruct((NC, NP, D), jnp.float32),
    mesh=_SC_MESH,
    scratch_types=[
        pltpu.VMEM_SHARED((NP, D), jnp.float32),
        pltpu.VMEM((CH,), jnp.int32),
        pltpu.VMEM((CH, D), jnp.float32),
    ],
)(_scatter_body)


def _logit_kernel(g0_ref, g1_ref, ea_ref, va_ref, vb_ref, ve_ref, c0_ref,
                  l_ref):
    s0 = jnp.sum(g0_ref[...] * va_ref[...], axis=1, keepdims=True)
    s1 = jnp.sum(g1_ref[...] * vb_ref[...], axis=1, keepdims=True)
    s2 = jnp.sum(ea_ref[...] * ve_ref[...], axis=1, keepdims=True)
    l_ref[...] = s0 + s1 + s2 + c0_ref[0, 0]


def _stats_kernel(l_ref, o_ref, sm):
    p = pl.program_id(0)
    j = pl.program_id(1)

    @pl.when(jnp.logical_and(p == 0, j == 0))
    def _():
        sm[0] = -3.0e38

    @pl.when(p == 0)
    def _():
        sm[0] = jnp.maximum(sm[0], jnp.max(l_ref[0, 0, :]))

    @pl.when(jnp.logical_and(p == 1, j == 0))
    def _():
        sm[1] = 0.0

    @pl.when(p == 1)
    def _():
        sm[1] = sm[1] + jnp.sum(jnp.exp(l_ref[0, 0, :] - sm[0]))

    o_ref[0, 0] = sm[0]
    o_ref[0, 1] = sm[1]


def _gelu(v):
    return 0.5 * v * (1.0 + lax.erf(v * 0.7071067811865476))


def _mlp_kernel(g0_ref, g1_ref, ea_ref, l_ref, st_ref,
                w1a_ref, w1b_ref, w1e_ref, b1_ref, w2_ref, b2_ref,
                w3_ref, b3_ref, o_ref):
    f32 = jnp.float32
    h = (jnp.dot(g0_ref[...], w1a_ref[...], preferred_element_type=f32)
         + jnp.dot(g1_ref[...], w1b_ref[...], preferred_element_type=f32)
         + jnp.dot(ea_ref[...], w1e_ref[...], preferred_element_type=f32)
         + b1_ref[...])
    h = _gelu(h)
    h = _gelu(jnp.dot(h, w2_ref[...], preferred_element_type=f32) + b2_ref[...])
    msg = jnp.dot(h, w3_ref[...], preferred_element_type=f32) + b3_ref[...]
    w = jnp.exp(l_ref[...] - st_ref[0, 0]) * (1.0 / st_ref[0, 1])
    o_ref[...] = w * msg


def _comb_kernel(p_ref, o_ref):
    o_ref[...] = p_ref[0, :N, :] + p_ref[1, :N, :]


def kernel(x, edge_attr, Wn, bn, We, be, Wa, ba, W1, b1, W2, b2, W3, b3,
           edge_index):
    f32 = jnp.float32
    row = edge_index[0]
    col = edge_index[1]

    wa1 = Wa[:H, 0]
    wa2 = Wa[H:2 * H, 0]
    wa3 = Wa[2 * H:, 0]
    va = (Wn @ wa1).reshape(1, D)
    vb = (Wn @ wa2).reshape(1, D)
    ve = (We @ wa3).reshape(1, DE)
    c0 = (bn @ wa1 + bn @ wa2 + be @ wa3 + ba[0]).reshape(1, 1)

    w1a = W1[:D]
    w1b = W1[D:2 * D]
    w1e = W1[2 * D:]

    g0, g1 = _gather_call(x, row, col)

    l = pl.pallas_call(
        _logit_kernel,
        grid=(NB,),
        in_specs=[
            pl.BlockSpec((MB, D), lambda j: (j, 0)),
            pl.BlockSpec((MB, D), lambda j: (j, 0)),
            pl.BlockSpec((MB, DE), lambda j: (j, 0)),
            pl.BlockSpec((1, D), lambda j: (0, 0)),
            pl.BlockSpec((1, D), lambda j: (0, 0)),
            pl.BlockSpec((1, DE), lambda j: (0, 0)),
            pl.BlockSpec((1, 1), lambda j: (0, 0), memory_space=pltpu.SMEM),
        ],
        out_specs=pl.BlockSpec((MB, 1), lambda j: (j, 0)),
        out_shape=jax.ShapeDtypeStruct((E, 1), f32),
    )(g0, g1, edge_attr, va, vb, ve, c0)

    stats = pl.pallas_call(
        _stats_kernel,
        grid=(2, NB),
        in_specs=[pl.BlockSpec((1, 1, MB), lambda p, j: (j, 0, 0))],
        out_specs=pl.BlockSpec((1, 2), lambda p, j: (0, 0),
                               memory_space=pltpu.SMEM),
        out_shape=jax.ShapeDtypeStruct((1, 2), f32),
        scratch_shapes=[pltpu.SMEM((2,), f32)],
    )(l.reshape(NB, 1, MB))

    wm = pl.pallas_call(
        _mlp_kernel,
        grid=(NB,),
        in_specs=[
            pl.BlockSpec((MB, D), lambda j: (j, 0)),
            pl.BlockSpec((MB, D), lambda j: (j, 0)),
            pl.BlockSpec((MB, DE), lambda j: (j, 0)),
            pl.BlockSpec((MB, 1), lambda j: (j, 0)),
            pl.BlockSpec((1, 2), lambda j: (0, 0), memory_space=pltpu.SMEM),
            pl.BlockSpec((D, H), lambda j: (0, 0)),
            pl.BlockSpec((D, H), lambda j: (0, 0)),
            pl.BlockSpec((DE, H), lambda j: (0, 0)),
            pl.BlockSpec((1, H), lambda j: (0, 0)),
            pl.BlockSpec((H, H), lambda j: (0, 0)),
            pl.BlockSpec((1, H), lambda j: (0, 0)),
            pl.BlockSpec((H, D), lambda j: (0, 0)),
            pl.BlockSpec((1, D), lambda j: (0, 0)),
        ],
        out_specs=pl.BlockSpec((MB, D), lambda j: (j, 0)),
        out_shape=jax.ShapeDtypeStruct((E, D), f32),
    )(g0, g1, edge_attr, l, stats, w1a, w1b, w1e, b1.reshape(1, H),
      W2, b2.reshape(1, H), W3, b3.reshape(1, D))

    parts = _scatter_call(wm, col, jnp.zeros((NP, D), f32))

    out = pl.pallas_call(
        _comb_kernel,
        out_shape=jax.ShapeDtypeStruct((N, D), f32),
    )(parts)

    return out


# speedup vs baseline: 4.4006x; 1.2107x over previous
import functools

import jax
import jax.numpy as jnp
from jax import lax
from jax.experimental import pallas as pl
from jax.experimental.pallas import tpu as pltpu
from jax.experimental.pallas import tpu_sc as plsc

N, E, D, DE, H = 10000, 320000, 128, 16, 256
NC, NS = 2, 16
NW = NC * NS
EPW = E // NW
CH = 80
NCH = EPW // CH
NP = 10240
NPT = NP // NS

MB = 2000
NB = E // MB

_SC_MESH = plsc.VectorSubcoreMesh(core_axis_name="c", subcore_axis_name="s")


def _gather_body(x_hbm, row_hbm, col_hbm, g0_hbm, g1_hbm,
                 rbuf, cbuf, r0a, r1a, r0b, r1b, s0a, s1a, s0b, s1b):
    c = lax.axis_index("c")
    s = lax.axis_index("s")
    wid = s * NC + c
    base = wid * EPW

    pltpu.sync_copy(row_hbm.at[pl.ds(base, EPW)], rbuf)
    pltpu.sync_copy(col_hbm.at[pl.ds(base, EPW)], cbuf)

    def fire(i, rows0, rows1, sg0, sg1):
        pltpu.async_copy(x_hbm.at[rbuf.at[pl.ds(i * CH, CH)]], rows0, sg0)
        pltpu.async_copy(x_hbm.at[cbuf.at[pl.ds(i * CH, CH)]], rows1, sg1)

    def gwait(rows0, rows1, sg0, sg1):
        pltpu.make_async_copy(x_hbm.at[rbuf.at[pl.ds(0, CH)]], rows0, sg0).wait()
        pltpu.make_async_copy(x_hbm.at[cbuf.at[pl.ds(0, CH)]], rows1, sg1).wait()

    def wback(i, rows0, rows1):
        e0 = base + i * CH
        pltpu.sync_copy(rows0, g0_hbm.at[pl.ds(e0, CH)])
        pltpu.sync_copy(rows1, g1_hbm.at[pl.ds(e0, CH)])

    fire(0, r0a, r1a, s0a, s1a)

    def body(k, carry):
        i = 2 * k
        fire(i + 1, r0b, r1b, s0b, s1b)
        gwait(r0a, r1a, s0a, s1a)
        wback(i, r0a, r1a)
        fire(i + 2, r0a, r1a, s0a, s1a)
        gwait(r0b, r1b, s0b, s1b)
        wback(i + 1, r0b, r1b)
        return carry

    lax.fori_loop(0, (NCH - 1) // 2, body, 0)
    gwait(r0a, r1a, s0a, s1a)
    wback(NCH - 1, r0a, r1a)


_gather_call = functools.partial(
    pl.kernel,
    out_type=(jax.ShapeDtypeStruct((E, D), jnp.float32),
              jax.ShapeDtypeStruct((E, D), jnp.float32)),
    mesh=_SC_MESH,
    scratch_types=[
        pltpu.VMEM((EPW,), jnp.int32),
        pltpu.VMEM((EPW,), jnp.int32),
        pltpu.VMEM((CH, D), jnp.float32),
        pltpu.VMEM((CH, D), jnp.float32),
        pltpu.VMEM((CH, D), jnp.float32),
        pltpu.VMEM((CH, D), jnp.float32),
        pltpu.SemaphoreType.DMA,
        pltpu.SemaphoreType.DMA,
        pltpu.SemaphoreType.DMA,
        pltpu.SemaphoreType.DMA,
    ],
)(_gather_body)


def _scatter_body(wm_hbm, col_hbm, zero_hbm, part_hbm, acc,
                  ia, wa, ib, wb, sia, swa, sib, swb):
    c = lax.axis_index("c")
    s = lax.axis_index("s")
    wid = s * NC + c
    base = wid * EPW
    r0 = s * NPT

    pltpu.sync_copy(zero_hbm.at[pl.ds(r0, NPT)], acc.at[pl.ds(r0, NPT)])
    plsc.subcore_barrier()

    def fire(i, ibuf, wbuf, si, sw):
        e0 = base + i * CH
        pltpu.async_copy(col_hbm.at[pl.ds(e0, CH)], ibuf, si)
        pltpu.async_copy(wm_hbm.at[pl.ds(e0, CH)], wbuf, sw)

    def lwait(ibuf, wbuf, si, sw):
        pltpu.make_async_copy(col_hbm.at[pl.ds(0, CH)], ibuf, si).wait()
        pltpu.make_async_copy(wm_hbm.at[pl.ds(0, CH)], wbuf, sw).wait()

    fire(0, ia, wa, sia, swa)

    def body(k, carry):
        i = 2 * k
        fire(i + 1, ib, wb, sib, swb)
        lwait(ia, wa, sia, swa)
        pltpu.sync_copy(wa, acc.at[ia], add=True)
        fire(i + 2, ia, wa, sia, swa)
        lwait(ib, wb, sib, swb)
        pltpu.sync_copy(wb, acc.at[ib], add=True)
        return carry

    lax.fori_loop(0, (NCH - 1) // 2, body, 0)
    lwait(ia, wa, sia, swa)
    pltpu.sync_copy(wa, acc.at[ia], add=True)

    plsc.subcore_barrier()
    pltpu.sync_copy(acc.at[pl.ds(r0, NPT)], part_hbm.at[c, pl.ds(r0, NPT)])


_scatter_call = functools.partial(
    pl.kernel,
    out_type=jax.ShapeDtypeStruct((NC, NP, D), jnp.float32),
    mesh=_SC_MESH,
    scratch_types=[
        pltpu.VMEM_SHARED((NP, D), jnp.float32),
        pltpu.VMEM((CH,), jnp.int32),
        pltpu.VMEM((CH, D), jnp.float32),
        pltpu.VMEM((CH,), jnp.int32),
        pltpu.VMEM((CH, D), jnp.float32),
        pltpu.SemaphoreType.DMA,
        pltpu.SemaphoreType.DMA,
        pltpu.SemaphoreType.DMA,
        pltpu.SemaphoreType.DMA,
    ],
)(_scatter_body)


def _logit_kernel(g0_ref, g1_ref, ea_ref, va_ref, vb_ref, ve_ref, c0_ref,
                  l_ref):
    s0 = jnp.sum(g0_ref[...] * va_ref[...], axis=1, keepdims=True)
    s1 = jnp.sum(g1_ref[...] * vb_ref[...], axis=1, keepdims=True)
    s2 = jnp.sum(ea_ref[...] * ve_ref[...], axis=1, keepdims=True)
    l_ref[...] = s0 + s1 + s2 + c0_ref[0, 0]


def _stats_kernel(l_ref, o_ref, sm):
    p = pl.program_id(0)
    j = pl.program_id(1)

    @pl.when(jnp.logical_and(p == 0, j == 0))
    def _():
        sm[0] = -3.0e38

    @pl.when(p == 0)
    def _():
        sm[0] = jnp.maximum(sm[0], jnp.max(l_ref[0, 0, :]))

    @pl.when(jnp.logical_and(p == 1, j == 0))
    def _():
        sm[1] = 0.0

    @pl.when(p == 1)
    def _():
        sm[1] = sm[1] + jnp.sum(jnp.exp(l_ref[0, 0, :] - sm[0]))

    o_ref[0, 0] = sm[0]
    o_ref[0, 1] = sm[1]


def _gelu(v):
    return 0.5 * v * (1.0 + lax.erf(v * 0.7071067811865476))


def _mlp_kernel(g0_ref, g1_ref, ea_ref, l_ref, st_ref,
                w1a_ref, w1b_ref, w1e_ref, b1_ref, w2_ref, b2_ref,
                w3_ref, b3_ref, o_ref):
    f32 = jnp.float32
    h = (jnp.dot(g0_ref[...], w1a_ref[...], preferred_element_type=f32)
         + jnp.dot(g1_ref[...], w1b_ref[...], preferred_element_type=f32)
         + jnp.dot(ea_ref[...], w1e_ref[...], preferred_element_type=f32)
         + b1_ref[...])
    h = _gelu(h)
    h = _gelu(jnp.dot(h, w2_ref[...], preferred_element_type=f32) + b2_ref[...])
    msg = jnp.dot(h, w3_ref[...], preferred_element_type=f32) + b3_ref[...]
    w = jnp.exp(l_ref[...] - st_ref[0, 0]) * (1.0 / st_ref[0, 1])
    o_ref[...] = w * msg


def _comb_kernel(p_ref, o_ref):
    o_ref[...] = p_ref[0, :N, :] + p_ref[1, :N, :]


def kernel(x, edge_attr, Wn, bn, We, be, Wa, ba, W1, b1, W2, b2, W3, b3,
           edge_index):
    f32 = jnp.float32
    row = edge_index[0]
    col = edge_index[1]

    wa1 = Wa[:H, 0]
    wa2 = Wa[H:2 * H, 0]
    wa3 = Wa[2 * H:, 0]
    va = (Wn @ wa1).reshape(1, D)
    vb = (Wn @ wa2).reshape(1, D)
    ve = (We @ wa3).reshape(1, DE)
    c0 = (bn @ wa1 + bn @ wa2 + be @ wa3 + ba[0]).reshape(1, 1)

    w1a = W1[:D]
    w1b = W1[D:2 * D]
    w1e = W1[2 * D:]

    g0, g1 = _gather_call(x, row, col)

    l = pl.pallas_call(
        _logit_kernel,
        grid=(NB,),
        in_specs=[
            pl.BlockSpec((MB, D), lambda j: (j, 0)),
            pl.BlockSpec((MB, D), lambda j: (j, 0)),
            pl.BlockSpec((MB, DE), lambda j: (j, 0)),
            pl.BlockSpec((1, D), lambda j: (0, 0)),
            pl.BlockSpec((1, D), lambda j: (0, 0)),
            pl.BlockSpec((1, DE), lambda j: (0, 0)),
            pl.BlockSpec((1, 1), lambda j: (0, 0), memory_space=pltpu.SMEM),
        ],
        out_specs=pl.BlockSpec((MB, 1), lambda j: (j, 0)),
        out_shape=jax.ShapeDtypeStruct((E, 1), f32),
    )(g0, g1, edge_attr, va, vb, ve, c0)

    stats = pl.pallas_call(
        _stats_kernel,
        grid=(2, NB),
        in_specs=[pl.BlockSpec((1, 1, MB), lambda p, j: (j, 0, 0))],
        out_specs=pl.BlockSpec((1, 2), lambda p, j: (0, 0),
                               memory_space=pltpu.SMEM),
        out_shape=jax.ShapeDtypeStruct((1, 2), f32),
        scratch_shapes=[pltpu.SMEM((2,), f32)],
    )(l.reshape(NB, 1, MB))

    wm = pl.pallas_call(
        _mlp_kernel,
        grid=(NB,),
        in_specs=[
            pl.BlockSpec((MB, D), lambda j: (j, 0)),
            pl.BlockSpec((MB, D), lambda j: (j, 0)),
            pl.BlockSpec((MB, DE), lambda j: (j, 0)),
            pl.BlockSpec((MB, 1), lambda j: (j, 0)),
            pl.BlockSpec((1, 2), lambda j: (0, 0), memory_space=pltpu.SMEM),
            pl.BlockSpec((D, H), lambda j: (0, 0)),
            pl.BlockSpec((D, H), lambda j: (0, 0)),
            pl.BlockSpec((DE, H), lambda j: (0, 0)),
            pl.BlockSpec((1, H), lambda j: (0, 0)),
            pl.BlockSpec((H, H), lambda j: (0, 0)),
            pl.BlockSpec((1, H), lambda j: (0, 0)),
            pl.BlockSpec((H, D), lambda j: (0, 0)),
            pl.BlockSpec((1, D), lambda j: (0, 0)),
        ],
        out_specs=pl.BlockSpec((MB, D), lambda j: (j, 0)),
        out_shape=jax.ShapeDtypeStruct((E, D), f32),
    )(g0, g1, edge_attr, l, stats, w1a, w1b, w1e, b1.reshape(1, H),
      W2, b2.reshape(1, H), W3, b3.reshape(1, D))

    parts = _scatter_call(wm, col, jnp.zeros((NP, D), f32))

    out = pl.pallas_call(
        _comb_kernel,
        out_shape=jax.ShapeDtypeStruct((N, D), f32),
    )(parts)

    return out
